# baseline (speedup 1.0000x reference)
import functools

import jax
import jax.numpy as jnp
from jax import lax
from jax.experimental import pallas as pl
from jax.experimental.pallas import tpu as pltpu
from jax.experimental.pallas import tpu_sc as plsc

BATCH = 16384
D = 128
NREL = 1000
RREP = 16
L = 16
NC, NS = 2, 16
NW = NC * NS
B_PER_W = BATCH // NW
C = 64
NCHUNK = B_PER_W // C
NBUF = 4
DCH = D // L


def _dist_mult_body(e1_hbm, r_hbm, e2_hbm, ent_hbm, rel_hbm, out_hbm,
                    i1_v, ir_v, i2_v,
                    b1a, bra, b2a, b1b, brb, b2b, b1c, brc, b2c,
                    b1d, brd, b2d,
                    tbuf, o_v, sem_a, sem_b, sem_c, sem_d, sem_i):
    cid = lax.axis_index("c")
    sid = lax.axis_index("s")
    wid = sid * NC + cid
    base = wid * B_PER_W
    row_iota = lax.iota(jnp.int32, L)

    bufs = ((b1a, bra, b2a), (b1b, brb, b2b), (b1c, brc, b2c),
            (b1d, brd, b2d))
    sems = (sem_a, sem_b, sem_c, sem_d)

    s_all = pl.ds(base, B_PER_W)
    ic1 = pltpu.async_copy(e1_hbm.at[s_all], i1_v, sem_i)
    ic2 = pltpu.async_copy(e2_hbm.at[s_all], i2_v, sem_i)
    icr = pltpu.async_copy(r_hbm.at[s_all], ir_v, sem_i)
    ic1.wait()
    ic2.wait()
    icr.wait()

    rep_off = (row_iota & (RREP - 1)) * NREL
    for t in range(B_PER_W // L):
        s = pl.ds(t * L, L)
        ir_v[s] = ir_v[s] + rep_off

    def fire(k):
        b1, br, b2 = bufs[k % NBUF]
        sem = sems[k % NBUF]
        s = pl.ds(k * C, C)
        return (pltpu.async_copy(ent_hbm.at[i1_v.at[s]], b1, sem),
                pltpu.async_copy(rel_hbm.at[ir_v.at[s]], br, sem),
                pltpu.async_copy(ent_hbm.at[i2_v.at[s]], b2, sem))

    def compute(k):
        b1, br, b2 = bufs[k % NBUF]

        @plsc.parallel_loop(0, C, step=1, unroll=4)
        def _rows(r):
            prods = []
            for j in range(DCH):
                s = pl.ds(j * L, L)
                prods.append(b1[r, s] * br[r, s] * b2[r, s])
            a0 = (prods[0] + prods[1]) + (prods[2] + prods[3])
            a1 = (prods[4] + prods[5]) + (prods[6] + prods[7])
            tbuf[pl.ds(r * (L + 1), L)] = a0 + a1

        @plsc.parallel_loop(0, C // L, step=1, unroll=2)
        def _groups(g):
            rows17 = (g * L + row_iota) * (L + 1)
            parts = [jnp.zeros((L,), jnp.float32) for _ in range(4)]
            for j in range(L):
                parts[j % 4] = parts[j % 4] + plsc.load_gather(
                    tbuf, [rows17 + j])
            acc = (parts[0] + parts[1]) + (parts[2] + parts[3])
            o_v[pl.ds(k * C + g * L, L)] = 1.0 / (1.0 + jnp.exp(-acc))

    inflight = {}
    for k in range(min(NBUF - 1, NCHUNK)):
        inflight[k] = fire(k)
    for k in range(NCHUNK):
        if k + NBUF - 1 < NCHUNK:
            inflight[k + NBUF - 1] = fire(k + NBUF - 1)
        for cp in inflight.pop(k):
            cp.wait()
        compute(k)
    pltpu.sync_copy(o_v, out_hbm.at[pl.ds(base, B_PER_W)])


@jax.jit
def _dist_mult(e1_idx, r_idx, e2_idx, entity_emb, relation_emb):
    rel_rep = jnp.tile(relation_emb, (RREP, 1))
    mesh = plsc.VectorSubcoreMesh(core_axis_name="c", subcore_axis_name="s")
    f = functools.partial(
        pl.kernel,
        mesh=mesh,
        compiler_params=pltpu.CompilerParams(needs_layout_passes=False),
        out_type=jax.ShapeDtypeStruct((BATCH,), jnp.float32),
        scratch_types=[
            pltpu.VMEM((B_PER_W,), jnp.int32),
            pltpu.VMEM((B_PER_W,), jnp.int32),
            pltpu.VMEM((B_PER_W,), jnp.int32),
            pltpu.VMEM((C, D), jnp.float32),
            pltpu.VMEM((C, D), jnp.float32),
            pltpu.VMEM((C, D), jnp.float32),
            pltpu.VMEM((C, D), jnp.float32),
            pltpu.VMEM((C, D), jnp.float32),
            pltpu.VMEM((C, D), jnp.float32),
            pltpu.VMEM((C, D), jnp.float32),
            pltpu.VMEM((C, D), jnp.float32),
            pltpu.VMEM((C, D), jnp.float32),
            pltpu.VMEM((C, D), jnp.float32),
            pltpu.VMEM((C, D), jnp.float32),
            pltpu.VMEM((C, D), jnp.float32),
            pltpu.VMEM((C * (L + 1),), jnp.float32),
            pltpu.VMEM((B_PER_W,), jnp.float32),
            pltpu.SemaphoreType.DMA,
            pltpu.SemaphoreType.DMA,
            pltpu.SemaphoreType.DMA,
            pltpu.SemaphoreType.DMA,
            pltpu.SemaphoreType.DMA,
        ],
    )(_dist_mult_body)
    return f(e1_idx, r_idx, e2_idx, entity_emb, rel_rep)


def kernel(e1_idx, r_idx, e2_idx, entity_emb, relation_emb):
    out = _dist_mult(e1_idx, r_idx, e2_idx, entity_emb, relation_emb)
    return (jnp.reshape(out, (-1,)), jnp.float32(0.0))


# speedup vs baseline: 1.0317x; 1.0317x over previous
import functools

import jax
import jax.numpy as jnp
from jax import lax
from jax.experimental import pallas as pl
from jax.experimental.pallas import tpu as pltpu
from jax.experimental.pallas import tpu_sc as plsc

BATCH = 16384
D = 128
NREL = 1000
RREP = 8
L = 16
NC, NS = 2, 16
NW = NC * NS
B_PER_W = BATCH // NW
C = 64
NCHUNK = B_PER_W // C
NBUF = 3
DCH = D // L


def _dist_mult_body(e1_hbm, r_hbm, e2_hbm, ent_hbm, rel_hbm, out_hbm,
                    i1_v, ir_v, i2_v,
                    b1a, bra, b2a, b1b, brb, b2b, b1c, brc, b2c,
                    tbuf, o_v, sem_a, sem_b, sem_c, sem_i):
    cid = lax.axis_index("c")
    sid = lax.axis_index("s")
    wid = sid * NC + cid
    base = wid * B_PER_W
    row_iota = lax.iota(jnp.int32, L)

    bufs = ((b1a, bra, b2a), (b1b, brb, b2b), (b1c, brc, b2c))
    sems = (sem_a, sem_b, sem_c)

    s_all = pl.ds(base, B_PER_W)
    ic1 = pltpu.async_copy(e1_hbm.at[s_all], i1_v, sem_i)
    ic2 = pltpu.async_copy(e2_hbm.at[s_all], i2_v, sem_i)
    icr = pltpu.async_copy(r_hbm.at[s_all], ir_v, sem_i)
    ic1.wait()
    ic2.wait()
    icr.wait()

    rep_off = (row_iota & (RREP - 1)) * NREL
    for t in range(B_PER_W // L):
        s = pl.ds(t * L, L)
        ir_v[s] = ir_v[s] + rep_off

    def fire(k):
        b1, br, b2 = bufs[k % NBUF]
        sem = sems[k % NBUF]
        s = pl.ds(k * C, C)
        return (pltpu.async_copy(ent_hbm.at[i1_v.at[s]], b1, sem),
                pltpu.async_copy(rel_hbm.at[ir_v.at[s]], br, sem),
                pltpu.async_copy(ent_hbm.at[i2_v.at[s]], b2, sem))

    def compute(k):
        b1, br, b2 = bufs[k % NBUF]

        @plsc.parallel_loop(0, C, step=1, unroll=4)
        def _rows(r):
            prods = []
            for j in range(DCH):
                s = pl.ds(j * L, L)
                prods.append(b1[r, s] * br[r, s] * b2[r, s])
            a0 = (prods[0] + prods[1]) + (prods[2] + prods[3])
            a1 = (prods[4] + prods[5]) + (prods[6] + prods[7])
            tbuf[pl.ds(r * (L + 1), L)] = a0 + a1

        @plsc.parallel_loop(0, C // L, step=1, unroll=2)
        def _groups(g):
            rows17 = (g * L + row_iota) * (L + 1)
            parts = [jnp.zeros((L,), jnp.float32) for _ in range(4)]
            for j in range(L):
                parts[j % 4] = parts[j % 4] + plsc.load_gather(
                    tbuf, [rows17 + j])
            acc = (parts[0] + parts[1]) + (parts[2] + parts[3])
            o_v[pl.ds(k * C + g * L, L)] = 1.0 / (1.0 + jnp.exp(-acc))

    inflight = {}
    for k in range(min(NBUF - 1, NCHUNK)):
        inflight[k] = fire(k)
    for k in range(NCHUNK):
        if k + NBUF - 1 < NCHUNK:
            inflight[k + NBUF - 1] = fire(k + NBUF - 1)
        for cp in inflight.pop(k):
            cp.wait()
        compute(k)
    pltpu.sync_copy(o_v, out_hbm.at[pl.ds(base, B_PER_W)])


@jax.jit
def _dist_mult(e1_idx, r_idx, e2_idx, entity_emb, relation_emb):
    rel_rep = jnp.tile(relation_emb, (RREP, 1))
    mesh = plsc.VectorSubcoreMesh(core_axis_name="c", subcore_axis_name="s")
    f = functools.partial(
        pl.kernel,
        mesh=mesh,
        compiler_params=pltpu.CompilerParams(needs_layout_passes=False),
        out_type=jax.ShapeDtypeStruct((BATCH,), jnp.float32),
        scratch_types=[
            pltpu.VMEM((B_PER_W,), jnp.int32),
            pltpu.VMEM((B_PER_W,), jnp.int32),
            pltpu.VMEM((B_PER_W,), jnp.int32),
            pltpu.VMEM((C, D), jnp.float32),
            pltpu.VMEM((C, D), jnp.float32),
            pltpu.VMEM((C, D), jnp.float32),
            pltpu.VMEM((C, D), jnp.float32),
            pltpu.VMEM((C, D), jnp.float32),
            pltpu.VMEM((C, D), jnp.float32),
            pltpu.VMEM((C, D), jnp.float32),
            pltpu.VMEM((C, D), jnp.float32),
            pltpu.VMEM((C, D), jnp.float32),
            pltpu.VMEM((C * (L + 1),), jnp.float32),
            pltpu.VMEM((B_PER_W,), jnp.float32),
            pltpu.SemaphoreType.DMA,
            pltpu.SemaphoreType.DMA,
            pltpu.SemaphoreType.DMA,
            pltpu.SemaphoreType.DMA,
        ],
    )(_dist_mult_body)
    return f(e1_idx, r_idx, e2_idx, entity_emb, rel_rep)


def kernel(e1_idx, r_idx, e2_idx, entity_emb, relation_emb):
    out = _dist_mult(e1_idx, r_idx, e2_idx, entity_emb, relation_emb)
    return (jnp.reshape(out, (-1,)), jnp.float32(0.0))
